# baseline (speedup 1.0000x reference)
import functools

import jax
import jax.numpy as jnp
import numpy as np
from jax import lax
from jax.experimental import pallas as pl
from jax.experimental.pallas import tpu as pltpu
from jax.experimental.pallas import tpu_sc as plsc

D_MODEL = 128
SEQ_LEN = 200
SCALE = float(np.sqrt(D_MODEL))
LANES = 16
NUM_WORKERS = 32
NBUF = 4
SB = 16
PB = 8
CHUNK = SB * PB


def _pe_table(dtype):
    p = jnp.arange(SEQ_LEN, dtype=jnp.float32)[:, None]
    i = jnp.arange(0, D_MODEL, 2, dtype=jnp.float32)
    ang = p / jnp.power(10000.0, i / D_MODEL)
    pe = jnp.zeros((SEQ_LEN, D_MODEL), dtype=jnp.float32)
    pe = pe.at[:, 0::2].set(jnp.sin(ang))
    pe = pe.at[:, 1::2].set(jnp.cos(ang))
    return pe.astype(dtype)


def _embed_kernel(batch):
    seqs_per_worker = batch // NUM_WORKERS
    sblocks = seqs_per_worker // SB
    nch = sblocks * (SEQ_LEN // PB)
    mesh = plsc.VectorSubcoreMesh(core_axis_name="c", subcore_axis_name="s")

    @functools.partial(
        pl.kernel,
        mesh=mesh,
        out_type=jax.ShapeDtypeStruct((batch * SEQ_LEN, D_MODEL), jnp.float32),
        scratch_types=[
            pltpu.VMEM((SEQ_LEN, D_MODEL), jnp.float32),
        ]
        + [pltpu.VMEM((CHUNK, D_MODEL), jnp.float32) for _ in range(NBUF)]
        + [pltpu.VMEM((CHUNK,), jnp.int32) for _ in range(NBUF)]
        + [pltpu.SemaphoreType.DMA for _ in range(3 * NBUF)],
    )
    def k(idx_hbm, table_hbm, pe_hbm, out_hbm, pe_v, *bufs_and_sems):
        rows = bufs_and_sems[:NBUF]
        glist = bufs_and_sems[NBUF : 2 * NBUF]
        gsem = bufs_and_sems[2 * NBUF : 3 * NBUF]
        ssem = bufs_and_sems[3 * NBUF : 4 * NBUF]
        fsem = bufs_and_sems[4 * NBUF : 5 * NBUF]

        wid = lax.axis_index("s") * 2 + lax.axis_index("c")
        seq0 = wid * seqs_per_worker
        pltpu.sync_copy(pe_hbm, pe_v)

        def fills(c, b):
            row0 = seq0 + (c % sblocks) * SB
            pb = c // sblocks
            return [
                pltpu.make_async_copy(
                    idx_hbm.at[pl.ds((row0 + s) * SEQ_LEN + pb * PB, PB)],
                    glist[b].at[pl.ds(s * PB, PB)],
                    fsem[b],
                )
                for s in range(SB)
            ]

        def start_fills(c, b):
            for cp in fills(c, b):
                cp.start()

        def wait_fills(c, b):
            for cp in fills(c, b):
                cp.wait()

        def gather(c, b):
            return pltpu.make_async_copy(table_hbm.at[glist[b]], rows[b], gsem[b])

        def stores(c, b):
            out0 = (seq0 + (c % sblocks) * SB) * SEQ_LEN + (c // sblocks) * PB
            return [
                pltpu.make_async_copy(
                    rows[b].at[pl.ds(s * PB, PB)],
                    out_hbm.at[pl.ds(out0 + s * SEQ_LEN, PB)],
                    ssem[b],
                )
                for s in range(SB)
            ]

        def start_stores(c, b):
            for cp in stores(c, b):
                cp.start()

        def wait_stores(c, b):
            for cp in stores(c, b):
                cp.wait()

        def compute_chunk(c, b):
            pe0 = (c // sblocks) * PB

            def p_body(p, carry):
                pr = pe0 + p
                for j in range(D_MODEL // LANES):
                    sl = pl.ds(j * LANES, LANES)
                    pej = pe_v[pr, sl]
                    for s in range(SB):
                        r = s * PB + p
                        rows[b][r, sl] = rows[b][r, sl] * SCALE + pej
                return carry

            lax.fori_loop(0, PB, p_body, 0)

        for b in range(NBUF):
            start_fills(b, b)
        for b in range(NBUF - 1):
            wait_fills(b, b)
            gather(b, b).start()

        assert (nch - NBUF) % NBUF == 0
        nmain = (nch - NBUF) // NBUF
        def main_body(i, carry):
            for j in range(NBUF):
                c = i * NBUF + j
                bp = (j + NBUF - 1) % NBUF
                gather(c, j).wait()
                start_fills(c + NBUF, j)
                if j == 0:
                    @pl.when(i > 0)
                    def _():
                        wait_stores(c - 1, bp)
                else:
                    wait_stores(c - 1, bp)
                wait_fills(c + NBUF - 1, bp)
                gather(c + NBUF - 1, bp).start()
                compute_chunk(c, j)
                start_stores(c, j)
            return carry

        lax.fori_loop(0, nmain, main_body, 0)

        for c in range(nch - NBUF, nch):
            b = c % NBUF
            gather(c, b).wait()
            wait_stores(c - 1, (c - 1) % NBUF)
            if c == nch - NBUF:
                wait_fills(nch - 1, (nch - 1) % NBUF)
                gather(nch - 1, (nch - 1) % NBUF).start()
            compute_chunk(c, b)
            start_stores(c, b)
        wait_stores(nch - 1, (nch - 1) % NBUF)

    return k


def kernel(input_x, table):
    batch, seq_len = input_x.shape
    assert seq_len == SEQ_LEN and table.shape[1] == D_MODEL
    idx3 = input_x.astype(jnp.int32).reshape(-1)
    pe = _pe_table(table.dtype)
    out = _embed_kernel(batch)(idx3, table, pe)
    return out.reshape(batch, seq_len, D_MODEL)


# speedup vs baseline: 1.0055x; 1.0055x over previous
import functools

import jax
import jax.numpy as jnp
import numpy as np
from jax import lax
from jax.experimental import pallas as pl
from jax.experimental.pallas import tpu as pltpu
from jax.experimental.pallas import tpu_sc as plsc

D_MODEL = 128
SEQ_LEN = 200
SCALE = float(np.sqrt(D_MODEL))
LANES = 16
NUM_WORKERS = 32
NBUF = 3
SB = 16
PB = 8
CHUNK = SB * PB


def _pe_table(dtype):
    p = jnp.arange(SEQ_LEN, dtype=jnp.float32)[:, None]
    i = jnp.arange(0, D_MODEL, 2, dtype=jnp.float32)
    ang = p / jnp.power(10000.0, i / D_MODEL)
    pe = jnp.zeros((SEQ_LEN, D_MODEL), dtype=jnp.float32)
    pe = pe.at[:, 0::2].set(jnp.sin(ang))
    pe = pe.at[:, 1::2].set(jnp.cos(ang))
    return pe.astype(dtype)


def _embed_kernel(batch):
    seqs_per_worker = batch // NUM_WORKERS
    sblocks = seqs_per_worker // SB
    nch = sblocks * (SEQ_LEN // PB)
    mesh = plsc.VectorSubcoreMesh(core_axis_name="c", subcore_axis_name="s")

    @functools.partial(
        pl.kernel,
        mesh=mesh,
        out_type=jax.ShapeDtypeStruct((batch * SEQ_LEN, D_MODEL), jnp.float32),
        scratch_types=[
            pltpu.VMEM((SEQ_LEN, D_MODEL), jnp.float32),
        ]
        + [pltpu.VMEM((CHUNK, D_MODEL), jnp.float32) for _ in range(NBUF)]
        + [pltpu.VMEM((CHUNK,), jnp.int32) for _ in range(NBUF)]
        + [pltpu.SemaphoreType.DMA for _ in range(3 * NBUF)],
    )
    def k(idx_hbm, table_hbm, pe_hbm, out_hbm, pe_v, *bufs_and_sems):
        rows = bufs_and_sems[:NBUF]
        glist = bufs_and_sems[NBUF : 2 * NBUF]
        gsem = bufs_and_sems[2 * NBUF : 3 * NBUF]
        ssem = bufs_and_sems[3 * NBUF : 4 * NBUF]
        fsem = bufs_and_sems[4 * NBUF : 5 * NBUF]

        wid = lax.axis_index("s") * 2 + lax.axis_index("c")
        seq0 = wid * seqs_per_worker
        pltpu.sync_copy(pe_hbm, pe_v)

        def fills(c, b):
            row0 = seq0 + (c % sblocks) * SB
            pb = c // sblocks
            return [
                pltpu.make_async_copy(
                    idx_hbm.at[pl.ds((row0 + s) * SEQ_LEN + pb * PB, PB)],
                    glist[b].at[pl.ds(s * PB, PB)],
                    fsem[b],
                )
                for s in range(SB)
            ]

        def start_fills(c, b):
            for cp in fills(c, b):
                cp.start()

        def wait_fills(c, b):
            for cp in fills(c, b):
                cp.wait()

        def gather(c, b):
            return pltpu.make_async_copy(table_hbm.at[glist[b]], rows[b], gsem[b])

        def stores(c, b):
            out0 = (seq0 + (c % sblocks) * SB) * SEQ_LEN + (c // sblocks) * PB
            return [
                pltpu.make_async_copy(
                    rows[b].at[pl.ds(s * PB, PB)],
                    out_hbm.at[pl.ds(out0 + s * SEQ_LEN, PB)],
                    ssem[b],
                )
                for s in range(SB)
            ]

        def start_stores(c, b):
            for cp in stores(c, b):
                cp.start()

        def wait_stores(c, b):
            for cp in stores(c, b):
                cp.wait()

        def compute_chunk(c, b):
            pe0 = (c // sblocks) * PB

            def p_body(p, carry):
                pr = pe0 + p
                for j in range(D_MODEL // LANES):
                    sl = pl.ds(j * LANES, LANES)
                    pej = pe_v[pr, sl]
                    for s in range(SB):
                        r = s * PB + p
                        rows[b][r, sl] = rows[b][r, sl] * SCALE + pej
                return carry

            lax.fori_loop(0, PB, p_body, 0)

        start_fills(0, 0)
        start_fills(1, 1)
        start_fills(2, 2)
        wait_fills(0, 0)
        gather(0, 0).start()
        wait_fills(1, 1)
        gather(1, 1).start()

        nmain = nch // NBUF
        def main_body(i, carry):
            for j in range(NBUF):
                c = i * NBUF + j
                b2 = (j + 2) % NBUF
                gather(c, j).wait()
                if j == NBUF - 1:
                    @pl.when(i < nmain - 1)
                    def _():
                        start_fills(c + 3, j)
                else:
                    start_fills(c + 3, j)
                if j == 0:
                    @pl.when(i > 0)
                    def _():
                        wait_stores(c - 1, b2)
                else:
                    wait_stores(c - 1, b2)
                wait_fills(c + 2, b2)
                gather(c + 2, b2).start()
                compute_chunk(c, j)
                start_stores(c, j)
            return carry

        lax.fori_loop(0, nmain, main_body, 0)

        for c in range(nmain * NBUF, nch):
            b = c % NBUF
            gather(c, b).wait()
            wait_stores(c - 1, (c - 1) % NBUF)
            compute_chunk(c, b)
            start_stores(c, b)
        wait_stores(nch - 1, (nch - 1) % NBUF)

    return k


def kernel(input_x, table):
    batch, seq_len = input_x.shape
    assert seq_len == SEQ_LEN and table.shape[1] == D_MODEL
    idx3 = input_x.astype(jnp.int32).reshape(-1)
    pe = _pe_table(table.dtype)
    out = _embed_kernel(batch)(idx3, table, pe)
    return out.reshape(batch, seq_len, D_MODEL)


# speedup vs baseline: 1.0072x; 1.0017x over previous
import functools

import jax
import jax.numpy as jnp
import numpy as np
from jax import lax
from jax.experimental import pallas as pl
from jax.experimental.pallas import tpu as pltpu
from jax.experimental.pallas import tpu_sc as plsc

D_MODEL = 128
SEQ_LEN = 200
SCALE = float(np.sqrt(D_MODEL))
LANES = 16
NUM_WORKERS = 32
NBUF = 3
SB = 16
PB = 8
CHUNK = SB * PB


def _pe_table(dtype):
    p = jnp.arange(SEQ_LEN, dtype=jnp.float32)[:, None]
    i = jnp.arange(0, D_MODEL, 2, dtype=jnp.float32)
    ang = p / jnp.power(10000.0, i / D_MODEL)
    pe = jnp.zeros((SEQ_LEN, D_MODEL), dtype=jnp.float32)
    pe = pe.at[:, 0::2].set(jnp.sin(ang))
    pe = pe.at[:, 1::2].set(jnp.cos(ang))
    return pe.astype(dtype)


def _embed_kernel(batch):
    seqs_per_worker = batch // NUM_WORKERS
    sblocks = seqs_per_worker // SB
    nch = sblocks * (SEQ_LEN // PB)
    mesh = plsc.VectorSubcoreMesh(core_axis_name="c", subcore_axis_name="s")

    @functools.partial(
        pl.kernel,
        mesh=mesh,
        out_type=jax.ShapeDtypeStruct((batch * SEQ_LEN, D_MODEL), jnp.float32),
        scratch_types=[
            pltpu.VMEM((SEQ_LEN, D_MODEL), jnp.float32),
        ]
        + [pltpu.VMEM((CHUNK, D_MODEL), jnp.float32) for _ in range(NBUF)]
        + [pltpu.VMEM((CHUNK,), jnp.int32) for _ in range(NBUF)]
        + [pltpu.SemaphoreType.DMA for _ in range(3 * NBUF)],
    )
    def k(idx_hbm, table_hbm, pe_hbm, out_hbm, pe_v, *bufs_and_sems):
        rows = bufs_and_sems[:NBUF]
        glist = bufs_and_sems[NBUF : 2 * NBUF]
        gsem = bufs_and_sems[2 * NBUF : 3 * NBUF]
        ssem = bufs_and_sems[3 * NBUF : 4 * NBUF]
        fsem = bufs_and_sems[4 * NBUF : 5 * NBUF]

        wid = lax.axis_index("s") * 2 + lax.axis_index("c")
        seq0 = wid * seqs_per_worker

        def fills(c, b):
            row0 = seq0 + (c % sblocks) * SB
            pb = c // sblocks
            return [
                pltpu.make_async_copy(
                    idx_hbm.at[pl.ds((row0 + s) * SEQ_LEN + pb * PB, PB)],
                    glist[b].at[pl.ds(s * PB, PB)],
                    fsem[b],
                )
                for s in range(SB)
            ]

        def start_fills(c, b):
            for cp in fills(c, b):
                cp.start()

        def wait_fills(c, b):
            for cp in fills(c, b):
                cp.wait()

        def gather(c, b):
            return pltpu.make_async_copy(table_hbm.at[glist[b]], rows[b], gsem[b])

        def stores(c, b):
            out0 = (seq0 + (c % sblocks) * SB) * SEQ_LEN + (c // sblocks) * PB
            return [
                pltpu.make_async_copy(
                    rows[b].at[pl.ds(s * PB, PB)],
                    out_hbm.at[pl.ds(out0 + s * SEQ_LEN, PB)],
                    ssem[b],
                )
                for s in range(SB)
            ]

        def start_stores(c, b):
            for cp in stores(c, b):
                cp.start()

        def wait_stores(c, b):
            for cp in stores(c, b):
                cp.wait()

        def compute_chunk(c, b):
            pe0 = (c // sblocks) * PB

            def p_body(p, carry):
                pr = pe0 + p
                for j in range(D_MODEL // LANES):
                    sl = pl.ds(j * LANES, LANES)
                    pej = pe_v[pr, sl]
                    for s in range(SB):
                        r = s * PB + p
                        rows[b][r, sl] = rows[b][r, sl] * SCALE + pej
                return carry

            lax.fori_loop(0, PB, p_body, 0)

        start_fills(0, 0)
        start_fills(1, 1)
        start_fills(2, 2)
        pltpu.sync_copy(pe_hbm, pe_v)
        wait_fills(0, 0)
        gather(0, 0).start()
        wait_fills(1, 1)
        gather(1, 1).start()

        nmain = nch // NBUF
        def main_body(i, carry):
            for j in range(NBUF):
                c = i * NBUF + j
                b2 = (j + 2) % NBUF
                gather(c, j).wait()
                if j == NBUF - 1:
                    @pl.when(i < nmain - 1)
                    def _():
                        start_fills(c + 3, j)
                else:
                    start_fills(c + 3, j)
                if j == 0:
                    @pl.when(i > 0)
                    def _():
                        wait_stores(c - 1, b2)
                else:
                    wait_stores(c - 1, b2)
                wait_fills(c + 2, b2)
                gather(c + 2, b2).start()
                compute_chunk(c, j)
                start_stores(c, j)
            return carry

        lax.fori_loop(0, nmain, main_body, 0)

        for c in range(nmain * NBUF, nch):
            b = c % NBUF
            gather(c, b).wait()
            wait_stores(c - 1, (c - 1) % NBUF)
            compute_chunk(c, b)
            start_stores(c, b)
        wait_stores(nch - 1, (nch - 1) % NBUF)

    return k


def kernel(input_x, table):
    batch, seq_len = input_x.shape
    assert seq_len == SEQ_LEN and table.shape[1] == D_MODEL
    idx3 = input_x.astype(jnp.int32).reshape(-1)
    pe = _pe_table(table.dtype)
    out = _embed_kernel(batch)(idx3, table, pe)
    return out.reshape(batch, seq_len, D_MODEL)
